# baseline (speedup 1.0000x reference)
import jax
import jax.numpy as jnp
from jax.experimental import pallas as pl


def _row_block(m, target):
    for bm in (target, 400, 200, 100, 40, 16, 8):
        if bm <= target and m % bm == 0:
            return bm
    return m


def _ln_relu(h, g, be):
    mu = jnp.mean(h, axis=-1, keepdims=True)
    d = h - mu
    var = jnp.mean(d * d, axis=-1, keepdims=True)
    return jnp.maximum(d * jax.lax.rsqrt(var + 1e-5) * g + be, 0.0)


def _bf(x):
    return x.astype(jnp.bfloat16)


def _feat_kernel(x_ref, w_ref, o_ref):
    o_ref[...] = _bf(jnp.dot(x_ref[...], w_ref[...],
                             preferred_element_type=jnp.float32))


def _pass1_kernel(adj_ref, y_ref, b_ref, g_ref, be_ref, w_ref, o_ref,
                  aq_ref, sinv_ref):
    a = adj_ref[...]
    rowmax = jnp.max(a, axis=1, keepdims=True)
    aq_ref[...] = jnp.round(a * (127.0 / rowmax)).astype(jnp.int8)
    sinv_ref[...] = rowmax * (1.0 / 127.0)
    h = jnp.dot(_bf(a), y_ref[...], preferred_element_type=jnp.float32)
    h = _ln_relu(h + b_ref[...], g_ref[...], be_ref[...])
    o_ref[...] = _bf(jnp.dot(h, w_ref[...],
                             preferred_element_type=jnp.float32))


def _tail_kernel(adj_ref, sinv_ref, y2_ref, b2_ref, g2_ref, be2_ref, w3_ref,
                 b3_ref, g3_ref, be3_ref, wf1_ref, bf1_ref, wf2_ref, bf2_ref,
                 o_ref, hin_s, y3_s, sum_s, max_s):
    p = pl.program_id(0)
    i = pl.program_id(1)
    nsteps = pl.num_programs(1)
    bm = adj_ref.shape[0]
    m = hin_s.shape[0]
    a = _bf(adj_ref[...])
    sinv = sinv_ref[...]

    @pl.when(p == 0)
    def _layer2():
        h = jnp.dot(a, y2_ref[...], preferred_element_type=jnp.float32)
        h = _ln_relu(h * sinv + b2_ref[...], g2_ref[...], be2_ref[...])
        hin_s[pl.ds(i * bm, bm), :] = h
        y3_s[pl.ds(i * bm, bm), :] = _bf(jnp.dot(
            h, w3_ref[...], preferred_element_type=jnp.float32))

    @pl.when(p == 1)
    def _layer3():
        h = jnp.dot(a, y3_s[...], preferred_element_type=jnp.float32)
        h = _ln_relu(h * sinv + b3_ref[...], g3_ref[...], be3_ref[...])
        h3 = h + hin_s[pl.ds(i * bm, bm), :]
        bsum = jnp.sum(h3, axis=0, keepdims=True)
        bmax = jnp.max(h3, axis=0, keepdims=True)

        @pl.when(i == 0)
        def _init():
            sum_s[...] = bsum
            max_s[...] = bmax

        @pl.when(i > 0)
        def _acc():
            sum_s[...] = sum_s[...] + bsum
            max_s[...] = jnp.maximum(max_s[...], bmax)

        @pl.when(i == nsteps - 1)
        def _head():
            gr = jnp.concatenate([sum_s[...] * (1.0 / m), max_s[...]], axis=1)
            out = jnp.maximum(
                jnp.dot(gr, wf1_ref[...], preferred_element_type=jnp.float32)
                + bf1_ref[...], 0.0)
            o_ref[...] = (jnp.dot(out, wf2_ref[...],
                                  preferred_element_type=jnp.float32)
                          + bf2_ref[...])


def _full(shape):
    return pl.BlockSpec(shape, lambda i: (0,) * len(shape))


def kernel(adj, features, W1, b1, g1, be1, W2, b2, g2, be2, W3, b3, g3, be3,
           Wf1, bf1, Wf2, bf2):
    m, n = adj.shape
    d_in = features.shape[1]
    c1 = W1.shape[1]
    c2 = W2.shape[1]
    c3 = W3.shape[1]
    bm = _row_block(m, 400)
    bmq = _row_block(m, 1000)

    b1r, g1r, be1r = b1[None, :], g1[None, :], be1[None, :]
    b2r, g2r, be2r = b2[None, :], g2[None, :], be2[None, :]
    b3r, g3r, be3r = b3[None, :], g3[None, :], be3[None, :]

    y1 = pl.pallas_call(
        _feat_kernel,
        grid=(m // bm,),
        in_specs=[pl.BlockSpec((bm, d_in), lambda i: (i, 0)),
                  _full((d_in, c1))],
        out_specs=pl.BlockSpec((bm, c1), lambda i: (i, 0)),
        out_shape=jax.ShapeDtypeStruct((m, c1), jnp.bfloat16),
    )(features, W1)

    rows = lambda b, c: pl.BlockSpec((b, c), lambda i: (i, 0))

    y2, adj_q, sinv = pl.pallas_call(
        _pass1_kernel,
        grid=(m // bm,),
        in_specs=[rows(bm, n), _full((n, c1)), _full((1, c1)), _full((1, c1)),
                  _full((1, c1)), _full((c1, c2))],
        out_specs=[rows(bm, c2), rows(bm, n), rows(bm, 1)],
        out_shape=[jax.ShapeDtypeStruct((m, c2), jnp.bfloat16),
                   jax.ShapeDtypeStruct((m, n), jnp.int8),
                   jax.ShapeDtypeStruct((m, 1), jnp.float32)],
    )(adj, y1, b1r, g1r, be1r, W2)

    nc = Wf2.shape[1]
    from jax.experimental.pallas import tpu as pltpu
    rows2 = lambda b, c: pl.BlockSpec((b, c), lambda p, i: (i, 0))
    pin = lambda shape: pl.BlockSpec(shape, lambda p, i: (0,) * len(shape))
    logits = pl.pallas_call(
        _tail_kernel,
        grid=(2, m // bmq),
        in_specs=[rows2(bmq, n), rows2(bmq, 1), pin((m, c2)), pin((1, c2)),
                  pin((1, c2)), pin((1, c2)), pin((c2, c3)),
                  pin((1, c3)), pin((1, c3)), pin((1, c3)),
                  pin((2 * c3, Wf1.shape[1])), pin((1, Wf1.shape[1])),
                  pin((Wf1.shape[1], nc)), pin((1, nc))],
        out_specs=pin((1, nc)),
        out_shape=jax.ShapeDtypeStruct((1, nc), jnp.float32),
        scratch_shapes=[pltpu.VMEM((m, c2), jnp.float32),
                        pltpu.VMEM((m, c3), jnp.bfloat16),
                        pltpu.VMEM((1, c3), jnp.float32),
                        pltpu.VMEM((1, c3), jnp.float32)],
    )(adj_q, sinv, y2, b2r, g2r, be2r, W3, b3r, g3r, be3r,
      Wf1, bf1[None, :], Wf2, bf2[None, :])

    return logits


# speedup vs baseline: 1.0579x; 1.0579x over previous
import jax
import jax.numpy as jnp
from jax.experimental import pallas as pl


def _row_block(m, target):
    for bm in (target, 400, 200, 100, 40, 16, 8):
        if bm <= target and m % bm == 0:
            return bm
    return m


def _ln_relu(h, g, be):
    mu = jnp.mean(h, axis=-1, keepdims=True)
    d = h - mu
    var = jnp.mean(d * d, axis=-1, keepdims=True)
    return jnp.maximum(d * jax.lax.rsqrt(var + 1e-5) * g + be, 0.0)


def _bf(x):
    return x.astype(jnp.bfloat16)


def _feat_kernel(x_ref, w_ref, o_ref):
    o_ref[...] = _bf(jnp.dot(x_ref[...], w_ref[...],
                             preferred_element_type=jnp.float32))


def _pass1_kernel(adj_ref, y_ref, b_ref, g_ref, be_ref, w_ref, o_ref,
                  aq_ref, sinv_ref):
    a = adj_ref[...]
    rowmax = jnp.max(a, axis=1, keepdims=True)
    aq_ref[...] = jnp.round(a * (127.0 / rowmax)).astype(jnp.int8)
    sinv_ref[...] = rowmax * (1.0 / 127.0)
    h = jnp.dot(_bf(a), y_ref[...], preferred_element_type=jnp.float32)
    h = _ln_relu(h + b_ref[...], g_ref[...], be_ref[...])
    o_ref[...] = _bf(jnp.dot(h, w_ref[...],
                             preferred_element_type=jnp.float32))


def _pass2_kernel(adj_ref, sinv_ref, y_ref, b_ref, g_ref, be_ref, w_ref,
                  h_ref, y3_ref):
    h = jnp.dot(_bf(adj_ref[...]), y_ref[...],
                preferred_element_type=jnp.float32)
    h = _ln_relu(h * sinv_ref[...] + b_ref[...], g_ref[...], be_ref[...])
    h_ref[...] = h
    y3_ref[...] = _bf(jnp.dot(h, w_ref[...],
                              preferred_element_type=jnp.float32))


def _pass3_kernel(adj_ref, sinv_ref, y_ref, b_ref, g_ref, be_ref, hin_ref,
                  wf1_ref, bf1_ref, wf2_ref, bf2_ref, o_ref, sum_s, max_s,
                  total_rows):
    i = pl.program_id(0)
    nsteps = pl.num_programs(0)
    h = jnp.dot(_bf(adj_ref[...]), y_ref[...],
                preferred_element_type=jnp.float32)
    h = _ln_relu(h * sinv_ref[...] + b_ref[...], g_ref[...], be_ref[...])
    h3 = h + hin_ref[...]
    bsum = jnp.sum(h3, axis=0, keepdims=True)
    bmax = jnp.max(h3, axis=0, keepdims=True)

    @pl.when(i == 0)
    def _init():
        sum_s[...] = bsum
        max_s[...] = bmax

    @pl.when(i > 0)
    def _acc():
        sum_s[...] = sum_s[...] + bsum
        max_s[...] = jnp.maximum(max_s[...], bmax)

    @pl.when(i == nsteps - 1)
    def _head():
        gr = jnp.concatenate([sum_s[...] * (1.0 / total_rows), max_s[...]],
                             axis=1)
        out = jnp.maximum(
            jnp.dot(gr, wf1_ref[...], preferred_element_type=jnp.float32)
            + bf1_ref[...], 0.0)
        o_ref[...] = (jnp.dot(out, wf2_ref[...],
                              preferred_element_type=jnp.float32)
                      + bf2_ref[...])


def _full(shape):
    return pl.BlockSpec(shape, lambda i: (0,) * len(shape))


def kernel(adj, features, W1, b1, g1, be1, W2, b2, g2, be2, W3, b3, g3, be3,
           Wf1, bf1, Wf2, bf2):
    m, n = adj.shape
    d_in = features.shape[1]
    c1 = W1.shape[1]
    c2 = W2.shape[1]
    c3 = W3.shape[1]
    bm = _row_block(m, 400)
    bmq = _row_block(m, 1000)

    b1r, g1r, be1r = b1[None, :], g1[None, :], be1[None, :]
    b2r, g2r, be2r = b2[None, :], g2[None, :], be2[None, :]
    b3r, g3r, be3r = b3[None, :], g3[None, :], be3[None, :]

    y1 = pl.pallas_call(
        _feat_kernel,
        grid=(m // bm,),
        in_specs=[pl.BlockSpec((bm, d_in), lambda i: (i, 0)),
                  _full((d_in, c1))],
        out_specs=pl.BlockSpec((bm, c1), lambda i: (i, 0)),
        out_shape=jax.ShapeDtypeStruct((m, c1), jnp.bfloat16),
    )(features, W1)

    rows = lambda b, c: pl.BlockSpec((b, c), lambda i: (i, 0))

    y2, adj_q, sinv = pl.pallas_call(
        _pass1_kernel,
        grid=(m // bm,),
        in_specs=[rows(bm, n), _full((n, c1)), _full((1, c1)), _full((1, c1)),
                  _full((1, c1)), _full((c1, c2))],
        out_specs=[rows(bm, c2), rows(bm, n), rows(bm, 1)],
        out_shape=[jax.ShapeDtypeStruct((m, c2), jnp.bfloat16),
                   jax.ShapeDtypeStruct((m, n), jnp.int8),
                   jax.ShapeDtypeStruct((m, 1), jnp.float32)],
    )(adj, y1, b1r, g1r, be1r, W2)

    h_in, y3 = pl.pallas_call(
        _pass2_kernel,
        grid=(m // bmq,),
        in_specs=[rows(bmq, n), rows(bmq, 1), _full((m, c2)), _full((1, c2)),
                  _full((1, c2)), _full((1, c2)), _full((c2, c3))],
        out_specs=[rows(bmq, c2), rows(bmq, c3)],
        out_shape=[jax.ShapeDtypeStruct((m, c2), jnp.float32),
                   jax.ShapeDtypeStruct((m, c3), jnp.bfloat16)],
    )(adj_q, sinv, y2, b2r, g2r, be2r, W3)

    import functools
    from jax.experimental.pallas import tpu as pltpu
    nc = Wf2.shape[1]
    logits = pl.pallas_call(
        functools.partial(_pass3_kernel, total_rows=m),
        grid=(m // bmq,),
        in_specs=[rows(bmq, n), rows(bmq, 1), _full((m, c3)), _full((1, c3)),
                  _full((1, c3)), _full((1, c3)), rows(bmq, c2),
                  _full((2 * c3, Wf1.shape[1])), _full((1, Wf1.shape[1])),
                  _full((Wf1.shape[1], nc)), _full((1, nc))],
        out_specs=_full((1, nc)),
        out_shape=jax.ShapeDtypeStruct((1, nc), jnp.float32),
        scratch_shapes=[pltpu.VMEM((1, c3), jnp.float32),
                        pltpu.VMEM((1, c3), jnp.float32)],
    )(adj_q, sinv, y3, b3r, g3r, be3r, h_in,
      Wf1, bf1[None, :], Wf2, bf2[None, :])

    return logits


# speedup vs baseline: 1.1074x; 1.0468x over previous
import jax
import jax.numpy as jnp
from jax.experimental import pallas as pl


def _row_block(m, target):
    for bm in (target, 400, 200, 100, 40, 16, 8):
        if bm <= target and m % bm == 0:
            return bm
    return m


def _ln_relu(h, g, be):
    mu = jnp.mean(h, axis=-1, keepdims=True)
    d = h - mu
    var = jnp.mean(d * d, axis=-1, keepdims=True)
    return jnp.maximum(d * jax.lax.rsqrt(var + 1e-5) * g + be, 0.0)


def _bf(x):
    return x.astype(jnp.bfloat16)


def _feat_kernel(x_ref, w_ref, o_ref):
    o_ref[...] = _bf(jnp.dot(x_ref[...], w_ref[...],
                             preferred_element_type=jnp.float32))


def _pass1_kernel(x_ref, w1_ref, adj_ref, b_ref, g_ref, be_ref, w_ref, o_ref,
                  aq_ref, sinv_ref, y_ref):
    @pl.when(pl.program_id(0) == 0)
    def _feat():
        y_ref[...] = _bf(jnp.dot(x_ref[...], w1_ref[...],
                                 preferred_element_type=jnp.float32))

    a = adj_ref[...]
    rowmax = jnp.max(a, axis=1, keepdims=True)
    aq_ref[...] = jnp.round(a * (127.0 / rowmax)).astype(jnp.int8)
    sinv_ref[...] = rowmax * (1.0 / 127.0)
    h = jnp.dot(_bf(a), y_ref[...], preferred_element_type=jnp.float32)
    h = _ln_relu(h + b_ref[...], g_ref[...], be_ref[...])
    o_ref[...] = _bf(jnp.dot(h, w_ref[...],
                             preferred_element_type=jnp.float32))


def _pass2_kernel(adj_ref, sinv_ref, y_ref, b_ref, g_ref, be_ref, w_ref,
                  h_ref, y3_ref):
    h = jnp.dot(_bf(adj_ref[...]), y_ref[...],
                preferred_element_type=jnp.float32)
    h = _ln_relu(h * sinv_ref[...] + b_ref[...], g_ref[...], be_ref[...])
    h_ref[...] = h
    y3_ref[...] = _bf(jnp.dot(h, w_ref[...],
                              preferred_element_type=jnp.float32))


def _pass3_kernel(adj_ref, sinv_ref, y_ref, b_ref, g_ref, be_ref, hin_ref,
                  wf1_ref, bf1_ref, wf2_ref, bf2_ref, o_ref, sum_s, max_s,
                  total_rows):
    i = pl.program_id(0)
    nsteps = pl.num_programs(0)
    h = jnp.dot(_bf(adj_ref[...]), y_ref[...],
                preferred_element_type=jnp.float32)
    h = _ln_relu(h * sinv_ref[...] + b_ref[...], g_ref[...], be_ref[...])
    h3 = h + hin_ref[...]
    bsum = jnp.sum(h3, axis=0, keepdims=True)
    bmax = jnp.max(h3, axis=0, keepdims=True)

    @pl.when(i == 0)
    def _init():
        sum_s[...] = bsum
        max_s[...] = bmax

    @pl.when(i > 0)
    def _acc():
        sum_s[...] = sum_s[...] + bsum
        max_s[...] = jnp.maximum(max_s[...], bmax)

    @pl.when(i == nsteps - 1)
    def _head():
        gr = jnp.concatenate([sum_s[...] * (1.0 / total_rows), max_s[...]],
                             axis=1)
        out = jnp.maximum(
            jnp.dot(gr, wf1_ref[...], preferred_element_type=jnp.float32)
            + bf1_ref[...], 0.0)
        o_ref[...] = (jnp.dot(out, wf2_ref[...],
                              preferred_element_type=jnp.float32)
                      + bf2_ref[...])


def _full(shape):
    return pl.BlockSpec(shape, lambda i: (0,) * len(shape))


def kernel(adj, features, W1, b1, g1, be1, W2, b2, g2, be2, W3, b3, g3, be3,
           Wf1, bf1, Wf2, bf2):
    m, n = adj.shape
    d_in = features.shape[1]
    c1 = W1.shape[1]
    c2 = W2.shape[1]
    c3 = W3.shape[1]
    bm = _row_block(m, 400)
    bmq = _row_block(m, 1000)

    b1r, g1r, be1r = b1[None, :], g1[None, :], be1[None, :]
    b2r, g2r, be2r = b2[None, :], g2[None, :], be2[None, :]
    b3r, g3r, be3r = b3[None, :], g3[None, :], be3[None, :]

    rows = lambda b, c: pl.BlockSpec((b, c), lambda i: (i, 0))

    from jax.experimental.pallas import tpu as pltpu
    y2, adj_q, sinv = pl.pallas_call(
        _pass1_kernel,
        grid=(m // bm,),
        in_specs=[_full((m, d_in)), _full((d_in, c1)), rows(bm, n),
                  _full((1, c1)), _full((1, c1)), _full((1, c1)),
                  _full((c1, c2))],
        out_specs=[rows(bm, c2), rows(bm, n), rows(bm, 1)],
        out_shape=[jax.ShapeDtypeStruct((m, c2), jnp.bfloat16),
                   jax.ShapeDtypeStruct((m, n), jnp.int8),
                   jax.ShapeDtypeStruct((m, 1), jnp.float32)],
        scratch_shapes=[pltpu.VMEM((n, c1), jnp.bfloat16)],
    )(features, W1, adj, b1r, g1r, be1r, W2)

    h_in, y3 = pl.pallas_call(
        _pass2_kernel,
        grid=(m // bmq,),
        in_specs=[rows(bmq, n), rows(bmq, 1), _full((m, c2)), _full((1, c2)),
                  _full((1, c2)), _full((1, c2)), _full((c2, c3))],
        out_specs=[rows(bmq, c2), rows(bmq, c3)],
        out_shape=[jax.ShapeDtypeStruct((m, c2), jnp.float32),
                   jax.ShapeDtypeStruct((m, c3), jnp.bfloat16)],
    )(adj_q, sinv, y2, b2r, g2r, be2r, W3)

    import functools
    from jax.experimental.pallas import tpu as pltpu
    nc = Wf2.shape[1]
    logits = pl.pallas_call(
        functools.partial(_pass3_kernel, total_rows=m),
        grid=(m // bmq,),
        in_specs=[rows(bmq, n), rows(bmq, 1), _full((m, c3)), _full((1, c3)),
                  _full((1, c3)), _full((1, c3)), rows(bmq, c2),
                  _full((2 * c3, Wf1.shape[1])), _full((1, Wf1.shape[1])),
                  _full((Wf1.shape[1], nc)), _full((1, nc))],
        out_specs=_full((1, nc)),
        out_shape=jax.ShapeDtypeStruct((1, nc), jnp.float32),
        scratch_shapes=[pltpu.VMEM((1, c3), jnp.float32),
                        pltpu.VMEM((1, c3), jnp.float32)],
    )(adj_q, sinv, y3, b3r, g3r, be3r, h_in,
      Wf1, bf1[None, :], Wf2, bf2[None, :])

    return logits
